# Initial kernel scaffold; baseline (speedup 1.0000x reference)
import jax
import jax.numpy as jnp
from jax.experimental import pallas as pl


def kernel(x, y, weight):
    raise NotImplementedError("write your pallas kernel here")


# speedup vs baseline: 2.8759x; 2.8759x over previous
import functools

import jax
import jax.numpy as jnp
from jax import lax
from jax.experimental import pallas as pl
from jax.experimental.pallas import tpu as pltpu
from jax.experimental.pallas import tpu_sc as plsc

NUM_EMB = 1_000_000
DIM = 16
BATCH = 16384
HIST = 50
N = BATCH * HIST

NC = 2
NS = 16
NW = NC * NS
N_PER_W = N // NW

SUB = 128
CHUNK = 1024
NSUB = CHUNK // SUB
NGROUP = CHUNK // 16
N_CHUNKS = N_PER_W // CHUNK


def _softmax_t(e):
    m = e[0]
    for v in e[1:]:
        m = jnp.maximum(m, v)
    u = [jnp.exp(v - m) for v in e]
    s = u[0]
    for v in u[1:]:
        s = s + v
    inv = 1.0 / s
    return [v * inv for v in u]


def _tree_l1(d):
    t4 = d[4] + d[13] + d[14] + d[15]
    t3 = d[3] + d[10] + d[11] + d[12]
    t2 = d[2] + d[7] + d[8] + d[9]
    t1 = d[1] + d[5] + d[6] + t4
    t0 = d[0] + t1 + t2 + t3
    acc = jnp.abs(t0) + jnp.abs(t1) + jnp.abs(t2) + jnp.abs(t3) + jnp.abs(t4)
    for i in range(5, 16):
        acc = acc + jnp.abs(d[i])
    return acc


def _sc_body(x_hbm, y_hbm, w_hbm, out_hbm, xi_v, yi_v, xr_v, yr_v, o_v, sem):
    wid = lax.axis_index("s") * NC + lax.axis_index("c")
    row0_w = wid * (N_PER_W // SUB)

    def chunk_body(c, carry):
        row0 = row0_w + c * NSUB
        base = row0 * SUB
        pltpu.sync_copy(x_hbm.at[pl.ds(row0, NSUB)], xi_v)
        pltpu.sync_copy(y_hbm.at[pl.ds(row0, NSUB)], yi_v)
        copies = []
        for j in range(NSUB):
            copies.append(pltpu.async_copy(
                w_hbm.at[xi_v.at[j]], xr_v.at[pl.ds(j * SUB, SUB)], sem))
            copies.append(pltpu.async_copy(
                w_hbm.at[yi_v.at[j]], yr_v.at[pl.ds(j * SUB, SUB)], sem))
        for cp in copies:
            cp.wait()

        def group_body(g, carry2):
            rows = g * 16 + lax.iota(jnp.int32, 16)
            ex = [plsc.load_gather(xr_v, [rows, jnp.full((16,), i, jnp.int32)])
                  for i in range(DIM)]
            rx = _softmax_t(ex)
            ey = [plsc.load_gather(yr_v, [rows, jnp.full((16,), i, jnp.int32)])
                  for i in range(DIM)]
            ry = _softmax_t(ey)
            d = [a - b for a, b in zip(rx, ry)]
            plsc.store_scatter(o_v, [rows], _tree_l1(d))
            return carry2

        lax.fori_loop(0, NGROUP, group_body, 0, unroll=False)
        pltpu.sync_copy(o_v, out_hbm.at[pl.ds(base, CHUNK)])
        return carry

    lax.fori_loop(0, N_CHUNKS, chunk_body, 0, unroll=False)


@jax.jit
def _poincare_sc(x2d, y2d, weight):
    mesh = plsc.VectorSubcoreMesh(core_axis_name="c", subcore_axis_name="s")
    f = pl.kernel(
        _sc_body,
        mesh=mesh,
        out_type=jax.ShapeDtypeStruct((N,), jnp.float32),
        scratch_types=[
            pltpu.VMEM((NSUB, SUB), jnp.int32),
            pltpu.VMEM((NSUB, SUB), jnp.int32),
            pltpu.VMEM((CHUNK, DIM), jnp.float32),
            pltpu.VMEM((CHUNK, DIM), jnp.float32),
            pltpu.VMEM((CHUNK,), jnp.float32),
            pltpu.SemaphoreType.DMA,
        ],
        compiler_params=pltpu.CompilerParams(
            needs_layout_passes=False, use_tc_tiling_on_sc=False),
    )
    return f(x2d, y2d, weight)


def kernel(x, y, weight):
    x2d = x.reshape(N // SUB, SUB)
    y2d = y.reshape(N // SUB, SUB)
    out = _poincare_sc(x2d, y2d, weight)
    return out.reshape(BATCH, HIST)
